# baseline (speedup 1.0000x reference)
import jax
import jax.numpy as jnp
from jax import lax
from jax.experimental import pallas as pl
from jax.experimental.pallas import tpu as pltpu
from jax.experimental.pallas import tpu_sc as plsc

E = 8
TOP_K = 2
H = 1024
F = 1024
T = 2048
A = TOP_K * T
RBLK = 256
NBLK = A // RBLK + E
NROWS = NBLK * RBLK
LANES = 128
NC = 2
NS = 16
NW = NC * NS
APW = A // NW

_sc_mesh = plsc.VectorSubcoreMesh(
    core_axis_name="c", subcore_axis_name="s", num_cores=NC, num_subcores=NS
)
_sc_params = pltpu.CompilerParams(needs_layout_passes=False)


def _lane16():
    return lax.broadcasted_iota(jnp.int32, (16,), 0)


def _router_body(x_ref, rw_ref, eids_ref, wts_ref, hist_ref):
    rw = rw_ref[...]
    x = x_ref[...]
    logits = lax.dot_general(
        rw, x, (((1,), (1,)), ((), ())), preferred_element_type=jnp.float32
    )
    row = lax.broadcasted_iota(jnp.int32, logits.shape, 0)
    neg = jnp.float32(-1e30)
    logits = jnp.where(row < E, logits, neg)
    m = jnp.max(logits, axis=0, keepdims=True)
    ex = jnp.exp(logits - m)
    ex = jnp.where(row < E, ex, 0.0)
    scores = ex / jnp.sum(ex, axis=0, keepdims=True)
    big = jnp.int32(LANES)
    m1 = jnp.max(scores, axis=0, keepdims=True)
    i1 = jnp.min(jnp.where(scores == m1, row, big), axis=0, keepdims=True)
    sc2 = jnp.where(row == i1, neg, scores)
    m2 = jnp.max(sc2, axis=0, keepdims=True)
    i2 = jnp.min(jnp.where(sc2 == m2, row, big), axis=0, keepdims=True)
    denom = m1 + m2
    krow = lax.broadcasted_iota(jnp.int32, (E, T), 0)
    eids_ref[...] = jnp.where(
        krow == 0, jnp.broadcast_to(i1, (E, T)),
        jnp.where(krow == 1, jnp.broadcast_to(i2, (E, T)), 0),
    )
    wts_ref[...] = jnp.where(
        krow == 0, jnp.broadcast_to(m1 / denom, (E, T)),
        jnp.where(krow == 1, jnp.broadcast_to(m2 / denom, (E, T)), 0.0),
    )
    erow = lax.broadcasted_iota(jnp.int32, (E, T), 0)
    oh0 = (erow == jnp.broadcast_to(i1, (E, T))).astype(jnp.float32)
    oh1 = (erow == jnp.broadcast_to(i2, (E, T))).astype(jnp.float32)
    tw = lax.broadcasted_iota(jnp.int32, (T, NS), 0) // APW
    ww = lax.broadcasted_iota(jnp.int32, (T, NS), 1)
    sel = (tw == ww).astype(jnp.float32)
    h0 = lax.dot_general(oh0, sel, (((1,), (0,)), ((), ())),
                         preferred_element_type=jnp.float32)
    h1 = lax.dot_general(oh1, sel, (((1,), (0,)), ((), ())),
                         preferred_element_type=jnp.float32)
    h01 = jnp.concatenate([h0, h1], axis=1).astype(jnp.int32)
    hist_ref[...] = jnp.concatenate(
        [h01, jnp.zeros((E, LANES - NW), jnp.int32)], axis=1
    )


def _router(xf, rw_pad):
    return pl.pallas_call(
        _router_body,
        grid=(1,),
        in_specs=[
            pl.BlockSpec((T, H), lambda i: (0, 0)),
            pl.BlockSpec((LANES, H), lambda i: (0, 0)),
        ],
        out_specs=[
            pl.BlockSpec((E, T), lambda i: (0, 0)),
            pl.BlockSpec((E, T), lambda i: (0, 0)),
            pl.BlockSpec((E, LANES), lambda i: (0, 0)),
        ],
        out_shape=[
            jax.ShapeDtypeStruct((E, T), jnp.int32),
            jax.ShapeDtypeStruct((E, T), jnp.float32),
            jax.ShapeDtypeStruct((E, LANES), jnp.int32),
        ],
    )(xf, rw_pad)


GCH = 32
GQ = APW // GCH


def _dispatch_body(eids_hbm, hist_hbm, x_hbm, sortpos_hbm, bexp_hbm,
                   act_hbm, xs_hbm, ev, hb, posb2, bexb, actb,
                   b0, b1, gs0, gs1, ws0, ws1):
    cid = lax.axis_index("c")
    sid = lax.axis_index("s")
    wid = sid * NC + cid
    lane = _lane16()
    zeros16 = jnp.zeros((16,), jnp.int32)

    pltpu.sync_copy(eids_hbm.at[pl.ds(wid * APW, APW)], ev)
    pltpu.sync_copy(hist_hbm, hb)

    tot_s, base_s = [], []
    for e in range(E):
        he0 = hb[e, pl.ds(0, 16)]
        he1 = hb[e, pl.ds(16, 16)]
        tot_s.append(jnp.sum(he0) + jnp.sum(he1))
        base_s.append(
            jnp.sum(jnp.where(lane < wid, he0, 0))
            + jnp.sum(jnp.where(lane + 16 < wid, he1, 0))
        )
    total = zeros16
    for e in range(E):
        total = total + jnp.where(lane == e, tot_s[e], 0)
    padded = ((total + (RBLK - 1)) >> 8) << 8
    ex_off = plsc.cumsum(padded) - padded

    off_s = [jnp.sum(jnp.where(lane == e, ex_off, 0)) for e in range(E)]
    pad_s = [jnp.sum(jnp.where(lane == e, padded, 0)) for e in range(E)]
    cnt = [off_s[e] + base_s[e] for e in range(E)]

    nr = jnp.sum(padded)
    last_e = jnp.max(jnp.where(padded > 0, lane, 0))
    for v in range(2):
        b = lane + v * 16
        r0 = b * RBLK
        bx = zeros16
        for e in range(E):
            inside = (r0 >= off_s[e]) & (r0 < off_s[e] + pad_s[e])
            bx = bx + jnp.where(inside, e, 0)
        active = r0 < nr
        bexb[pl.ds(v * 16, 16)] = jnp.where(active, bx, last_e)
        actb[pl.ds(v * 16, 16)] = jnp.where(active, 1, 0)
    pltpu.sync_copy(bexb, bexp_hbm.at[wid])
    pltpu.sync_copy(actb, act_hbm.at[wid])

    for j in range(APW // 16):
        evj = ev[pl.ds(j * 16, 16)]
        pos = zeros16
        for e in range(E):
            mask = evj == e
            mi = jnp.where(mask, 1, 0)
            pref = plsc.cumsum(mi) - mi
            pos = jnp.where(mask, cnt[e] + pref, pos)
            cnt[e] = cnt[e] + jnp.sum(mi)
        posb2[j // (GCH // 16), pl.ds((j % (GCH // 16)) * 16, 16)] = pos

    bufs = [(b0, gs0, ws0), (b1, gs1, ws1)]
    rdesc = [None, None]
    wdesc = [None, None]

    def start(q):
        bb, gs, _ = bufs[q % 2]
        if wdesc[q % 2] is not None:
            wdesc[q % 2].wait()
        a0 = wid * APW + q * GCH
        t0 = pl.multiple_of(a0 & (T - 1), GCH)
        rdesc[q % 2] = pltpu.async_copy(x_hbm.at[pl.ds(t0, GCH)], bb, gs)

    start(0)
    for q in range(GQ):
        if q + 1 < GQ:
            start(q + 1)
        bb, _, ws = bufs[q % 2]
        pltpu.sync_copy(posb2.at[q],
                        sortpos_hbm.at[pl.ds(wid * APW + q * GCH, GCH)])
        rdesc[q % 2].wait()
        wdesc[q % 2] = pltpu.async_copy(bb, xs_hbm.at[posb2.at[q]], ws)
    wdesc[(GQ - 1) % 2].wait()
    wdesc[GQ % 2].wait()


_dispatch = pl.kernel(
    _dispatch_body,
    out_type=[
        jax.ShapeDtypeStruct((A,), jnp.int32),
        jax.ShapeDtypeStruct((NW, 32), jnp.int32),
        jax.ShapeDtypeStruct((NW, 32), jnp.int32),
        jax.ShapeDtypeStruct((NROWS, H), jnp.float32),
    ],
    mesh=_sc_mesh,
    compiler_params=_sc_params,
    scratch_types=[
        pltpu.VMEM((APW,), jnp.int32),
        pltpu.VMEM((E, LANES), jnp.int32),
        pltpu.VMEM((GQ, GCH), jnp.int32),
        pltpu.VMEM((32,), jnp.int32),
        pltpu.VMEM((32,), jnp.int32),
        pltpu.VMEM((GCH, H), jnp.float32),
        pltpu.VMEM((GCH, H), jnp.float32),
        pltpu.SemaphoreType.DMA,
        pltpu.SemaphoreType.DMA,
        pltpu.SemaphoreType.DMA,
        pltpu.SemaphoreType.DMA,
    ],
)


def _gemm_body(bexp_ref, act_ref, xs_ref, w1_ref, v1_ref, w2_ref, y_ref):
    b = pl.program_id(0)

    @pl.when(act_ref[b] > 0)
    def _():
        xb = xs_ref[...].astype(jnp.bfloat16)
        w1b = w1_ref[0].astype(jnp.bfloat16)
        v1b = v1_ref[0].astype(jnp.bfloat16)
        w2b = w2_ref[0].astype(jnp.bfloat16)
        h1 = lax.dot_general(
            xb, w1b, (((1,), (1,)), ((), ())), preferred_element_type=jnp.float32
        )
        h2 = lax.dot_general(
            xb, v1b, (((1,), (1,)), ((), ())), preferred_element_type=jnp.float32
        )
        h = (h1 * jax.nn.sigmoid(h1) * h2).astype(jnp.bfloat16)
        y_ref[...] = lax.dot_general(
            h, w2b, (((1,), (0,)), ((), ())), preferred_element_type=jnp.float32
        )


def _gemm(bexp, act, xs, w1, v1, w2):
    grid_spec = pltpu.PrefetchScalarGridSpec(
        num_scalar_prefetch=2,
        grid=(NBLK,),
        in_specs=[
            pl.BlockSpec((RBLK, H), lambda b, be, act: (act[b] * b, 0)),
            pl.BlockSpec((1, F, H), lambda b, be, act: (be[b], 0, 0)),
            pl.BlockSpec((1, F, H), lambda b, be, act: (be[b], 0, 0)),
            pl.BlockSpec((1, F, H), lambda b, be, act: (be[b], 0, 0)),
        ],
        out_specs=pl.BlockSpec((RBLK, H), lambda b, be, act: (b, 0)),
    )
    return pl.pallas_call(
        _gemm_body,
        grid_spec=grid_spec,
        out_shape=jax.ShapeDtypeStruct((NROWS, H), jnp.float32),
    )(bexp, act, xs, w1, v1, w2)


CCH = 16
CQ = T // CCH // NW


def _combine_body(y_hbm, sortpos_hbm, wts_hbm, out_hbm,
                  i0, i1, wb0, wb1, b0, b1, ob, gs0, gs1, osem):
    cid = lax.axis_index("c")
    sid = lax.axis_index("s")
    wid = sid * NC + cid
    lane = _lane16()
    bufs = [(i0, wb0, b0, gs0), (i1, wb1, b1, gs1)]
    gdesc = [None, None]

    def start(q):
        ib, wb, bb, gs = bufs[q % 2]
        t0 = (wid + q * NW) * CCH
        pltpu.sync_copy(sortpos_hbm.at[pl.ds(t0, CCH)], ib.at[pl.ds(0, CCH)])
        pltpu.sync_copy(sortpos_hbm.at[pl.ds(T + t0, CCH)],
                        ib.at[pl.ds(CCH, CCH)])
        pltpu.sync_copy(wts_hbm.at[0, pl.ds(t0, CCH)], wb.at[pl.ds(0, CCH)])
        pltpu.sync_copy(wts_hbm.at[1, pl.ds(t0, CCH)], wb.at[pl.ds(CCH, CCH)])
        gdesc[q % 2] = pltpu.async_copy(y_hbm.at[ib], bb, gs)

    start(0)
    for q in range(CQ):
        if q + 1 < CQ:
            start(q + 1)
        ib, wb, bb, gs = bufs[q % 2]
        gdesc[q % 2].wait()
        w0v = wb[pl.ds(0, 16)]
        w1v = wb[pl.ds(16, 16)]
        s0 = [jnp.sum(jnp.where(lane == i, w0v, 0.0)) for i in range(CCH)]
        s1 = [jnp.sum(jnp.where(lane == i, w1v, 0.0)) for i in range(CCH)]

        def body(c, _):
            sl = pl.ds(c * 16, 16)
            for i in range(CCH):
                ob[i, sl] = bb[i, sl] * s0[i] + bb[CCH + i, sl] * s1[i]
            return 0

        lax.fori_loop(0, H // 16, body, 0)
        t0 = (wid + q * NW) * CCH
        pltpu.sync_copy(ob, out_hbm.at[pl.ds(t0, CCH)])


_combine = pl.kernel(
    _combine_body,
    out_type=jax.ShapeDtypeStruct((T, H), jnp.float32),
    mesh=_sc_mesh,
    compiler_params=_sc_params,
    scratch_types=[
        pltpu.VMEM((2 * CCH,), jnp.int32),
        pltpu.VMEM((2 * CCH,), jnp.int32),
        pltpu.VMEM((2 * CCH,), jnp.float32),
        pltpu.VMEM((2 * CCH,), jnp.float32),
        pltpu.VMEM((2 * CCH, H), jnp.float32),
        pltpu.VMEM((2 * CCH, H), jnp.float32),
        pltpu.VMEM((CCH, H), jnp.float32),
        pltpu.SemaphoreType.DMA,
        pltpu.SemaphoreType.DMA,
        pltpu.SemaphoreType.DMA,
    ],
)


@jax.jit
def kernel(hidden_states, router_w, w1, v1, w2):
    xf = hidden_states.reshape(T, H)
    rw_pad = jnp.zeros((LANES, H), jnp.float32).at[:E].set(router_w)

    eids, wts, hist = _router(xf, rw_pad)
    eids_flat = eids[:TOP_K].reshape(A)
    sortpos, bexp, act, xs = _dispatch(eids_flat, hist, xf)
    y = _gemm(bexp[0, :NBLK], act[0, :NBLK], xs, w1, v1, w2)
    out = _combine(y, sortpos, wts[:TOP_K])
    return out.reshape(1, T, H)


# speedup vs baseline: 1.8640x; 1.8640x over previous
import jax
import jax.numpy as jnp
from jax import lax
from jax.experimental import pallas as pl
from jax.experimental.pallas import tpu as pltpu
from jax.experimental.pallas import tpu_sc as plsc

E = 8
TOP_K = 2
H = 1024
F = 1024
T = 2048
A = TOP_K * T
RBLK = 256
NBLK = A // RBLK + E
NROWS = NBLK * RBLK
LANES = 128
NC = 2
NS = 16
NW = NC * NS
APW = A // NW

_sc_mesh = plsc.VectorSubcoreMesh(
    core_axis_name="c", subcore_axis_name="s", num_cores=NC, num_subcores=NS
)
_sc_params = pltpu.CompilerParams(needs_layout_passes=False)


def _lane16():
    return lax.broadcasted_iota(jnp.int32, (16,), 0)


def _router_body(x_ref, rw_ref, eids_ref, wts_ref, hist_ref):
    rw = rw_ref[...]
    x = x_ref[...]
    logits = lax.dot_general(
        rw, x, (((1,), (1,)), ((), ())), preferred_element_type=jnp.float32
    )
    row = lax.broadcasted_iota(jnp.int32, logits.shape, 0)
    neg = jnp.float32(-1e30)
    logits = jnp.where(row < E, logits, neg)
    m = jnp.max(logits, axis=0, keepdims=True)
    ex = jnp.exp(logits - m)
    ex = jnp.where(row < E, ex, 0.0)
    scores = ex / jnp.sum(ex, axis=0, keepdims=True)
    big = jnp.int32(LANES)
    m1 = jnp.max(scores, axis=0, keepdims=True)
    i1 = jnp.min(jnp.where(scores == m1, row, big), axis=0, keepdims=True)
    sc2 = jnp.where(row == i1, neg, scores)
    m2 = jnp.max(sc2, axis=0, keepdims=True)
    i2 = jnp.min(jnp.where(sc2 == m2, row, big), axis=0, keepdims=True)
    denom = m1 + m2
    krow = lax.broadcasted_iota(jnp.int32, (E, T), 0)
    eids_ref[...] = jnp.where(
        krow == 0, jnp.broadcast_to(i1, (E, T)),
        jnp.where(krow == 1, jnp.broadcast_to(i2, (E, T)), 0),
    )
    wts_ref[...] = jnp.where(
        krow == 0, jnp.broadcast_to(m1 / denom, (E, T)),
        jnp.where(krow == 1, jnp.broadcast_to(m2 / denom, (E, T)), 0.0),
    )
    erow = lax.broadcasted_iota(jnp.int32, (E, T), 0)
    oh0 = (erow == jnp.broadcast_to(i1, (E, T))).astype(jnp.float32)
    oh1 = (erow == jnp.broadcast_to(i2, (E, T))).astype(jnp.float32)
    tw = lax.broadcasted_iota(jnp.int32, (T, NS), 0) // APW
    ww = lax.broadcasted_iota(jnp.int32, (T, NS), 1)
    sel = (tw == ww).astype(jnp.float32)
    h0 = lax.dot_general(oh0, sel, (((1,), (0,)), ((), ())),
                         preferred_element_type=jnp.float32)
    h1 = lax.dot_general(oh1, sel, (((1,), (0,)), ((), ())),
                         preferred_element_type=jnp.float32)
    h01 = jnp.concatenate([h0, h1], axis=1).astype(jnp.int32)
    hist_ref[...] = jnp.concatenate(
        [h01, jnp.zeros((E, LANES - NW), jnp.int32)], axis=1
    )


def _router(xf, rw_pad):
    return pl.pallas_call(
        _router_body,
        grid=(1,),
        in_specs=[
            pl.BlockSpec((T, H), lambda i: (0, 0)),
            pl.BlockSpec((LANES, H), lambda i: (0, 0)),
        ],
        out_specs=[
            pl.BlockSpec((E, T), lambda i: (0, 0)),
            pl.BlockSpec((E, T), lambda i: (0, 0)),
            pl.BlockSpec((E, LANES), lambda i: (0, 0)),
        ],
        out_shape=[
            jax.ShapeDtypeStruct((E, T), jnp.int32),
            jax.ShapeDtypeStruct((E, T), jnp.float32),
            jax.ShapeDtypeStruct((E, LANES), jnp.int32),
        ],
    )(xf, rw_pad)


GCH = 32
GQ = APW // GCH


def _dispatch_body(eids_hbm, hist_hbm, x_hbm, sortpos_hbm, bexp_hbm,
                   act_hbm, xs_hbm, ev, hb, posb2, bexb, actb,
                   b0, b1, gs0, gs1, ws0, ws1):
    cid = lax.axis_index("c")
    sid = lax.axis_index("s")
    wid = sid * NC + cid
    lane = _lane16()
    zeros16 = jnp.zeros((16,), jnp.int32)

    pltpu.sync_copy(eids_hbm.at[pl.ds(wid * APW, APW)], ev)
    pltpu.sync_copy(hist_hbm, hb)

    tot_s, base_s = [], []
    for e in range(E):
        he0 = hb[e, pl.ds(0, 16)]
        he1 = hb[e, pl.ds(16, 16)]
        tot_s.append(jnp.sum(he0) + jnp.sum(he1))
        base_s.append(
            jnp.sum(jnp.where(lane < wid, he0, 0))
            + jnp.sum(jnp.where(lane + 16 < wid, he1, 0))
        )
    total = zeros16
    for e in range(E):
        total = total + jnp.where(lane == e, tot_s[e], 0)
    padded = ((total + (RBLK - 1)) >> 8) << 8
    ex_off = plsc.cumsum(padded) - padded

    off_s = [jnp.sum(jnp.where(lane == e, ex_off, 0)) for e in range(E)]
    pad_s = [jnp.sum(jnp.where(lane == e, padded, 0)) for e in range(E)]
    cnt = [off_s[e] + base_s[e] for e in range(E)]

    nr = jnp.sum(padded)
    last_e = jnp.max(jnp.where(padded > 0, lane, 0))
    for v in range(2):
        b = lane + v * 16
        r0 = b * RBLK
        bx = zeros16
        for e in range(E):
            inside = (r0 >= off_s[e]) & (r0 < off_s[e] + pad_s[e])
            bx = bx + jnp.where(inside, e, 0)
        active = r0 < nr
        bexb[pl.ds(v * 16, 16)] = jnp.where(active, bx, last_e)
        actb[pl.ds(v * 16, 16)] = jnp.where(active, 1, 0)
    pltpu.sync_copy(bexb, bexp_hbm.at[wid])
    pltpu.sync_copy(actb, act_hbm.at[wid])

    for j in range(APW // 16):
        evj = ev[pl.ds(j * 16, 16)]
        pos = zeros16
        for e in range(E):
            mask = evj == e
            mi = jnp.where(mask, 1, 0)
            pref = plsc.cumsum(mi) - mi
            pos = jnp.where(mask, cnt[e] + pref, pos)
            cnt[e] = cnt[e] + jnp.sum(mi)
        posb2[j // (GCH // 16), pl.ds((j % (GCH // 16)) * 16, 16)] = pos

    bufs = [(b0, gs0, ws0), (b1, gs1, ws1)]
    rdesc = [None, None]
    wdesc = [None, None]

    def start(q):
        bb, gs, _ = bufs[q % 2]
        if wdesc[q % 2] is not None:
            wdesc[q % 2].wait()
        a0 = wid * APW + q * GCH
        t0 = pl.multiple_of(a0 & (T - 1), GCH)
        rdesc[q % 2] = pltpu.async_copy(x_hbm.at[pl.ds(t0, GCH)], bb, gs)

    start(0)
    for q in range(GQ):
        if q + 1 < GQ:
            start(q + 1)
        bb, _, ws = bufs[q % 2]
        pltpu.sync_copy(posb2.at[q],
                        sortpos_hbm.at[pl.ds(wid * APW + q * GCH, GCH)])
        rdesc[q % 2].wait()
        wdesc[q % 2] = pltpu.async_copy(bb, xs_hbm.at[posb2.at[q]], ws)
    wdesc[(GQ - 1) % 2].wait()
    wdesc[GQ % 2].wait()


_dispatch = pl.kernel(
    _dispatch_body,
    out_type=[
        jax.ShapeDtypeStruct((A,), jnp.int32),
        jax.ShapeDtypeStruct((NW, 32), jnp.int32),
        jax.ShapeDtypeStruct((NW, 32), jnp.int32),
        jax.ShapeDtypeStruct((NROWS, H), jnp.float32),
    ],
    mesh=_sc_mesh,
    compiler_params=_sc_params,
    scratch_types=[
        pltpu.VMEM((APW,), jnp.int32),
        pltpu.VMEM((E, LANES), jnp.int32),
        pltpu.VMEM((GQ, GCH), jnp.int32),
        pltpu.VMEM((32,), jnp.int32),
        pltpu.VMEM((32,), jnp.int32),
        pltpu.VMEM((GCH, H), jnp.float32),
        pltpu.VMEM((GCH, H), jnp.float32),
        pltpu.SemaphoreType.DMA,
        pltpu.SemaphoreType.DMA,
        pltpu.SemaphoreType.DMA,
        pltpu.SemaphoreType.DMA,
    ],
)


def _gemm_body(bexp_ref, act_ref, xs_ref, w1_ref, v1_ref, w2_ref, y_ref):
    b = pl.program_id(0)

    @pl.when(act_ref[b] > 0)
    def _():
        xb = xs_ref[...].astype(jnp.bfloat16)
        w1b = w1_ref[0].astype(jnp.bfloat16)
        v1b = v1_ref[0].astype(jnp.bfloat16)
        w2b = w2_ref[0].astype(jnp.bfloat16)
        h1 = lax.dot_general(
            xb, w1b, (((1,), (1,)), ((), ())), preferred_element_type=jnp.float32
        )
        h2 = lax.dot_general(
            xb, v1b, (((1,), (1,)), ((), ())), preferred_element_type=jnp.float32
        )
        h = (h1 * jax.nn.sigmoid(h1) * h2).astype(jnp.bfloat16)
        y_ref[...] = lax.dot_general(
            h, w2b, (((1,), (0,)), ((), ())), preferred_element_type=jnp.float32
        )


def _gemm(bexp, act, xs, w1, v1, w2):
    grid_spec = pltpu.PrefetchScalarGridSpec(
        num_scalar_prefetch=2,
        grid=(NBLK,),
        in_specs=[
            pl.BlockSpec((RBLK, H), lambda b, be, act: (act[b] * b, 0)),
            pl.BlockSpec((1, F, H), lambda b, be, act: (be[b], 0, 0)),
            pl.BlockSpec((1, F, H), lambda b, be, act: (be[b], 0, 0)),
            pl.BlockSpec((1, F, H), lambda b, be, act: (be[b], 0, 0)),
        ],
        out_specs=pl.BlockSpec((RBLK, H), lambda b, be, act: (b, 0)),
    )
    return pl.pallas_call(
        _gemm_body,
        grid_spec=grid_spec,
        out_shape=jax.ShapeDtypeStruct((NROWS, H), jnp.float32),
    )(bexp, act, xs, w1, v1, w2)


CCH = 16
CQ = T // CCH // NW


def _combine_body(y_hbm, sortpos_hbm, wts_hbm, out_hbm,
                  sp0, sp1, wv0, wv1, ibq, b0, b1, ob, gs0, gs1):
    cid = lax.axis_index("c")
    sid = lax.axis_index("s")
    wid = sid * NC + cid
    lane = _lane16()
    t0 = wid * (CQ * CCH)
    pltpu.sync_copy(sortpos_hbm.at[pl.ds(t0, CQ * CCH)], sp0)
    pltpu.sync_copy(sortpos_hbm.at[pl.ds(T + t0, CQ * CCH)], sp1)
    pltpu.sync_copy(wts_hbm.at[0, pl.ds(t0, CQ * CCH)], wv0)
    pltpu.sync_copy(wts_hbm.at[1, pl.ds(t0, CQ * CCH)], wv1)
    for q in range(CQ):
        ibq[q, pl.ds(0, CCH)] = sp0[pl.ds(q * CCH, CCH)]
        ibq[q, pl.ds(CCH, CCH)] = sp1[pl.ds(q * CCH, CCH)]

    bufs = [(b0, gs0), (b1, gs1)]
    gdesc = [None, None]

    def start(q):
        bb, gs = bufs[q % 2]
        gdesc[q % 2] = pltpu.async_copy(y_hbm.at[ibq.at[q]], bb, gs)

    start(0)
    for q in range(CQ):
        if q + 1 < CQ:
            start(q + 1)
        bb, _ = bufs[q % 2]
        gdesc[q % 2].wait()
        w0v = wv0[pl.ds(q * CCH, CCH)]
        w1v = wv1[pl.ds(q * CCH, CCH)]
        s0 = [jnp.sum(jnp.where(lane == i, w0v, 0.0)) for i in range(CCH)]
        s1 = [jnp.sum(jnp.where(lane == i, w1v, 0.0)) for i in range(CCH)]

        def body(c, _):
            sl = pl.ds(c * 16, 16)
            for i in range(CCH):
                ob[i, sl] = bb[i, sl] * s0[i] + bb[CCH + i, sl] * s1[i]
            return 0

        lax.fori_loop(0, H // 16, body, 0)
        pltpu.sync_copy(ob, out_hbm.at[pl.ds(t0 + q * CCH, CCH)])


_combine = pl.kernel(
    _combine_body,
    out_type=jax.ShapeDtypeStruct((T, H), jnp.float32),
    mesh=_sc_mesh,
    compiler_params=_sc_params,
    scratch_types=[
        pltpu.VMEM((CQ * CCH,), jnp.int32),
        pltpu.VMEM((CQ * CCH,), jnp.int32),
        pltpu.VMEM((CQ * CCH,), jnp.float32),
        pltpu.VMEM((CQ * CCH,), jnp.float32),
        pltpu.VMEM((CQ, 2 * CCH), jnp.int32),
        pltpu.VMEM((2 * CCH, H), jnp.float32),
        pltpu.VMEM((2 * CCH, H), jnp.float32),
        pltpu.VMEM((CCH, H), jnp.float32),
        pltpu.SemaphoreType.DMA,
        pltpu.SemaphoreType.DMA,
    ],
)


@jax.jit
def kernel(hidden_states, router_w, w1, v1, w2):
    xf = hidden_states.reshape(T, H)
    rw_pad = jnp.zeros((LANES, H), jnp.float32).at[:E].set(router_w)

    eids, wts, hist = _router(xf, rw_pad)
    eids_flat = eids[:TOP_K].reshape(A)
    sortpos, bexp, act, xs = _dispatch(eids_flat, hist, xf)
    y = _gemm(bexp[0, :NBLK], act[0, :NBLK], xs, w1, v1, w2)
    out = _combine(y, sortpos, wts[:TOP_K])
    return out.reshape(1, T, H)
